# baseline (speedup 1.0000x reference)
import functools

import jax
import jax.numpy as jnp
from jax import lax
from jax.experimental import pallas as pl
from jax.experimental.pallas import tpu as pltpu

E = 8
TOPK = 2
IN = 1024
RH = 512
RH2 = 256
H1 = 1024
H2 = 512
NC = 256
N = 2048
NB = 256
EP = 128
NT = N // NB
B = 256
NCH = (N * TOPK) // B + E - 1
NCHG = NCH + 1
CHA = 32
APAD = NCHG * B


def _router_body(x_ref, wr1_ref, br1_ref, wr2_ref, br2_ref, wr3_ref, br3_ref,
                 probs_ref, meta_ref, counts_ref, runc_ref):
    t = pl.program_id(0)
    x = x_ref[...]
    h = jnp.maximum(
        jnp.dot(x, wr1_ref[...], preferred_element_type=jnp.float32)
        + br1_ref[...], 0.0)
    h = jnp.maximum(
        jnp.dot(h, wr2_ref[...], preferred_element_type=jnp.float32)
        + br2_ref[...], 0.0)
    s = jnp.dot(h, wr3_ref[...], preferred_element_type=jnp.float32) \
        + br3_ref[...]
    lane = lax.broadcasted_iota(jnp.int32, (NB, EP), 1)
    s = jnp.where(lane < E, s, -1e30)
    m = jnp.max(s, axis=1, keepdims=True)
    p = jnp.exp(s - m)
    probs = p / jnp.sum(p, axis=1, keepdims=True)
    probs_ref[...] = probs
    v1 = jnp.max(probs, axis=1, keepdims=True)
    i1 = jnp.min(jnp.where(probs == v1, lane, EP), axis=1, keepdims=True)
    pm = jnp.where(lane == i1, -1.0, probs)
    v2 = jnp.max(pm, axis=1, keepdims=True)
    i2 = jnp.min(jnp.where(pm == v2, lane, EP), axis=1, keepdims=True)

    m0 = (lane == i1).astype(jnp.float32)
    m1 = (lane == i2).astype(jnp.float32)
    msum = m0 + m1
    ri = lax.broadcasted_iota(jnp.int32, (NB, NB), 0)
    ci = lax.broadcasted_iota(jnp.int32, (NB, NB), 1)
    tri = (ri > ci).astype(jnp.float32)
    within = jnp.dot(tri, msum, preferred_element_type=jnp.float32)

    @pl.when(t == 0)
    def _init():
        runc_ref[...] = jnp.zeros((1, EP), jnp.float32)

    runb = runc_ref[...]
    base = runb + within
    r0 = jnp.sum(jnp.where(lane == i1, base, 0.0), axis=1, keepdims=True)
    r1 = jnp.sum(jnp.where(lane == i2, base, 0.0), axis=1, keepdims=True)
    runc_ref[...] = runb + jnp.sum(msum, axis=0, keepdims=True)
    counts_ref[...] = runc_ref[...]

    meta = (jnp.where(lane == 0, v1 * 0.5, 0.0)
            + jnp.where(lane == 1, v2 * 0.5, 0.0)
            + jnp.where(lane == 2, i1.astype(jnp.float32), 0.0)
            + jnp.where(lane == 3, i2.astype(jnp.float32), 0.0)
            + jnp.where(lane == 4, r0, 0.0)
            + jnp.where(lane == 5, r1, 0.0))
    meta_ref[...] = meta


def _experts_body(ce_ref, cv_ref, xs_ref, we1_ref, be1_ref, we2_ref, be2_ref,
                  we3_ref, be3_ref, ys_ref):
    c = pl.program_id(0)

    @pl.when(cv_ref[c] == 1)
    def _compute():
        xb = xs_ref[...]
        h1 = jnp.maximum(
            jnp.dot(xb, we1_ref[0], preferred_element_type=jnp.float32)
            + be1_ref[0], 0.0)
        h2 = jnp.maximum(
            jnp.dot(h1, we2_ref[0], preferred_element_type=jnp.float32)
            + be2_ref[0], 0.0)
        ys_ref[...] = jnp.dot(
            h2, we3_ref[0], preferred_element_type=jnp.float32) + be3_ref[0]


def _combine_body(meta_ref, g0_ref, g1_ref, out_ref):
    lane = lax.broadcasted_iota(jnp.int32, (NB, EP), 1)
    meta = meta_ref[...]
    wa = jnp.sum(jnp.where(lane == 0, meta, 0.0), axis=1, keepdims=True)
    wb = jnp.sum(jnp.where(lane == 1, meta, 0.0), axis=1, keepdims=True)
    out_ref[...] = wa * g0_ref[...] + wb * g1_ref[...]


@jax.jit
def kernel(x, Wr1, br1, Wr2, br2, Wr3, br3, We1, be1, We2, be2, We3, be3):
    wr3p = jnp.pad(Wr3, ((0, 0), (0, EP - E)))
    br3p = jnp.pad(br3, (0, EP - E)).reshape(1, EP)

    probs_full, meta, counts = pl.pallas_call(
        _router_body,
        grid=(NT,),
        in_specs=[
            pl.BlockSpec((NB, IN), lambda t: (t, 0)),
            pl.BlockSpec((IN, RH), lambda t: (0, 0)),
            pl.BlockSpec((1, RH), lambda t: (0, 0)),
            pl.BlockSpec((RH, RH2), lambda t: (0, 0)),
            pl.BlockSpec((1, RH2), lambda t: (0, 0)),
            pl.BlockSpec((RH2, EP), lambda t: (0, 0)),
            pl.BlockSpec((1, EP), lambda t: (0, 0)),
        ],
        out_specs=[
            pl.BlockSpec((NB, EP), lambda t: (t, 0)),
            pl.BlockSpec((NB, EP), lambda t: (t, 0)),
            pl.BlockSpec((1, EP), lambda t: (0, 0)),
        ],
        out_shape=[
            jax.ShapeDtypeStruct((N, EP), jnp.float32),
            jax.ShapeDtypeStruct((N, EP), jnp.float32),
            jax.ShapeDtypeStruct((1, EP), jnp.float32),
        ],
        scratch_shapes=[pltpu.VMEM((1, EP), jnp.float32)],
        compiler_params=pltpu.CompilerParams(
            dimension_semantics=("arbitrary",)),
    )(x, Wr1, br1.reshape(1, RH), Wr2, br2.reshape(1, RH2), wr3p, br3p)

    i1 = meta[:, 2].astype(jnp.int32)
    i2 = meta[:, 3].astype(jnp.int32)
    r0 = meta[:, 4].astype(jnp.int32)
    r1 = meta[:, 5].astype(jnp.int32)
    cnt = counts[0, :E].astype(jnp.int32)
    cc = (cnt + (B - 1)) // B
    incl = jnp.cumsum(cc)
    pad_start = (incl - cc) * B
    j = jnp.arange(CHA, dtype=jnp.int32)
    acc = jnp.sum((j[:, None] >= incl[None, :]).astype(jnp.int32), axis=1)
    chunk_expert = jnp.minimum(acc, E - 1)
    chunk_valid = (acc < E).astype(jnp.int32)
    pos0 = pad_start[i1] + r0
    pos1 = pad_start[i2] + r1
    tok = jnp.arange(N, dtype=jnp.int32)
    st = jnp.zeros((APAD,), jnp.int32).at[pos0].set(tok).at[pos1].set(tok)

    xs = x[st]

    ys = pl.pallas_call(
        _experts_body,
        grid_spec=pltpu.PrefetchScalarGridSpec(
            num_scalar_prefetch=2,
            grid=(NCHG,),
            in_specs=[
                pl.BlockSpec((B, IN), lambda c, ce, cv: (c, 0)),
                pl.BlockSpec((1, IN, H1), lambda c, ce, cv: (ce[c], 0, 0)),
                pl.BlockSpec((1, 1, H1), lambda c, ce, cv: (ce[c], 0, 0)),
                pl.BlockSpec((1, H1, H2), lambda c, ce, cv: (ce[c], 0, 0)),
                pl.BlockSpec((1, 1, H2), lambda c, ce, cv: (ce[c], 0, 0)),
                pl.BlockSpec((1, H2, NC), lambda c, ce, cv: (ce[c], 0, 0)),
                pl.BlockSpec((1, 1, NC), lambda c, ce, cv: (ce[c], 0, 0)),
            ],
            out_specs=pl.BlockSpec((B, NC), lambda c, ce, cv: (c, 0)),
        ),
        out_shape=jax.ShapeDtypeStruct((APAD, NC), jnp.float32),
        compiler_params=pltpu.CompilerParams(
            dimension_semantics=("arbitrary",)),
    )(chunk_expert, chunk_valid, xs, We1, be1.reshape(E, 1, H1),
      We2, be2.reshape(E, 1, H2), We3, be3.reshape(E, 1, NC))

    g0 = ys[pos0]
    g1 = ys[pos1]

    out = pl.pallas_call(
        _combine_body,
        grid=(NT,),
        in_specs=[
            pl.BlockSpec((NB, EP), lambda t: (t, 0)),
            pl.BlockSpec((NB, NC), lambda t: (t, 0)),
            pl.BlockSpec((NB, NC), lambda t: (t, 0)),
        ],
        out_specs=pl.BlockSpec((NB, NC), lambda t: (t, 0)),
        out_shape=jax.ShapeDtypeStruct((N, NC), jnp.float32),
    )(meta, g0, g1)

    return out, probs_full[:, :E]
